# Initial kernel scaffold; baseline (speedup 1.0000x reference)
import jax
import jax.numpy as jnp
from jax.experimental import pallas as pl


def kernel(x, sparsity_matrix, norm_gamma, norm_beta, Wq, bq, Wk, bk, Wv, bv, Wp, bp):
    raise NotImplementedError("write your pallas kernel here")


# speedup vs baseline: 3.0154x; 3.0154x over previous
import jax
import jax.numpy as jnp
from jax.experimental import pallas as pl
from jax.experimental.pallas import tpu as pltpu

B, C, H, W = 4, 768, 32, 32
HEADS = 8
DH = C // HEADS
HW = H * W
GROUPS = 32
CPG = C // GROUPS
EPS = 1e-6
SCALE = float(DH) ** -0.5
N_GN = CPG * HW


def _attn_block_body(xT_ref, mask_ref, gamma_ref, beta_ref,
                     wq_ref, bq_ref, wk_ref, bk_ref, wv_ref, bv_ref,
                     wp_ref, bp_ref, out_ref, h2_ref):
    f32 = jnp.float32
    bf16 = jnp.bfloat16
    xT = xT_ref[0]

    gsel = (jax.lax.broadcasted_iota(jnp.int32, (C, C), 0) // CPG
            == jax.lax.broadcasted_iota(jnp.int32, (C, C), 1) // CPG).astype(f32)
    colsum = jnp.sum(xT, axis=0, keepdims=True)
    mean = jax.lax.dot_general(colsum, gsel, (((1,), (0,)), ((), ())),
                               preferred_element_type=f32) * (1.0 / N_GN)
    xc = xT - mean
    sq = jnp.sum(xc * xc, axis=0, keepdims=True)
    var = jax.lax.dot_general(sq, gsel, (((1,), (0,)), ((), ())),
                              preferred_element_type=f32) * (1.0 / N_GN)
    hb = xc * (jax.lax.rsqrt(var + EPS) * gamma_ref[...]) + beta_ref[...]
    hb16 = hb.astype(bf16)

    q16 = ((jnp.dot(hb16, wq_ref[...], preferred_element_type=f32)
            + bq_ref[...]) * SCALE).astype(bf16)
    k16 = (jnp.dot(hb16, wk_ref[...], preferred_element_type=f32)
           + bk_ref[...]).astype(bf16)
    v16 = (jnp.dot(hb16, wv_ref[...], preferred_element_type=f32)
           + bv_ref[...]).astype(bf16)

    mask = mask_ref[0]

    for h in range(HEADS):
        sl = slice(h * DH, (h + 1) * DH)
        qh = q16[:, sl]
        kh = k16[:, sl]
        vh = v16[:, sl]
        s = jax.lax.dot_general(qh, kh, (((1,), (1,)), ((), ())),
                                preferred_element_type=f32)
        mx = jnp.max(s, axis=1, keepdims=True)
        p = jnp.exp(s - mx) * mask
        denom = jnp.sum(p, axis=1, keepdims=True)
        oh = jnp.dot(p.astype(bf16), vh, preferred_element_type=f32)
        h2_ref[:, sl] = oh / denom
    h2_16 = h2_ref[...].astype(bf16)
    out = jnp.dot(h2_16, wp_ref[...], preferred_element_type=f32) + bp_ref[...]
    out_ref[0] = out + xT


def kernel(x, sparsity_matrix, norm_gamma, norm_beta,
           Wq, bq, Wk, bk, Wv, bv, Wp, bp):
    xT = x.reshape(B, C, HW).transpose(0, 2, 1)
    bf16 = jnp.bfloat16
    wq, wk, wv, wp = (w.T.astype(bf16) for w in (Wq, Wk, Wv, Wp))
    bq_r, bk_r, bv_r, bp_r = (b.reshape(1, C) for b in (bq, bk, bv, bp))
    gamma_r = norm_gamma.reshape(1, C)
    beta_r = norm_beta.reshape(1, C)

    full = lambda shape: pl.BlockSpec(shape, lambda i: (0,) * len(shape))
    outT = pl.pallas_call(
        _attn_block_body,
        grid=(B,),
        in_specs=[
            pl.BlockSpec((1, HW, C), lambda i: (i, 0, 0)),
            pl.BlockSpec((1, HW, HW), lambda i: (i, 0, 0)),
            full((1, C)), full((1, C)),
            full((C, C)), full((1, C)),
            full((C, C)), full((1, C)),
            full((C, C)), full((1, C)),
            full((C, C)), full((1, C)),
        ],
        out_specs=pl.BlockSpec((1, HW, C), lambda i: (i, 0, 0)),
        out_shape=jax.ShapeDtypeStruct((B, HW, C), jnp.float32),
        scratch_shapes=[pltpu.VMEM((HW, C), jnp.float32)],
    )(xT, sparsity_matrix, gamma_r, beta_r,
      wq, bq_r, wk, bk_r, wv, bv_r, wp, bp_r)
    out = outT.transpose(0, 2, 1).reshape(B, C, H, W)
    return (out, out)
